# baseline (speedup 1.0000x reference)
import functools

import jax
import jax.numpy as jnp
import numpy as np
from jax import lax
from jax.experimental import pallas as pl
from jax.experimental.pallas import tpu as pltpu
from jax.experimental.pallas import tpu_sc as plsc

BIG = np.float32(1e9)
K = 16
N = 4096
HALF = 128
LOSS0 = 32
STAGE0 = LOSS0 + 24 * 16


def _sort16(x, descending=False):
    k, _ = plsc.sort_key_val(x, x, descending=descending)
    return k


def _merge_top(u, l, xd):
    lo1 = jnp.minimum(u, xd)
    u2 = _sort16(jnp.maximum(u, xd), descending=False)
    hi2 = jnp.maximum(l, _sort16(lo1, descending=True))
    l2 = _sort16(hi2, descending=False)
    return u2, l2


def _merge_sets(u0, l0, u1, l1):
    u, l = _merge_top(u0, l0, _sort16(u1, descending=True))
    return _merge_top(u, l, _sort16(l1, descending=True))


def _valid_mask(e, st):
    inf = np.float32(np.inf)
    fin = (jnp.abs(e) != inf) & (jnp.abs(st) != inf)
    nz = (e * st) != np.float32(0.0)
    df = (e - st) != np.float32(0.0)
    return jnp.where(fin & nz & df, np.float32(1.0), np.float32(0.0))


def _sqrt16(xv):
    bits = plsc.bitcast(xv, jnp.int32)
    y = plsc.bitcast((bits >> 1) + np.int32(0x1FBD1DF5), jnp.float32)
    half = np.float32(0.5)
    for _ in range(4):
        y = half * (y + xv / y)
    return y


def _toploss_body(beta_hbm, ground_hbm, out_hbm, half_a, half_b, half_c,
                  s32_v, loss12_v, res_v):
    c = lax.axis_index("c")
    s = lax.axis_index("s")
    iota = lax.iota(jnp.int32, 16)

    halves = (half_a, half_b, half_c)
    q1 = s // 8
    sgn = jnp.where(q1 == 0, np.float32(1.0), np.float32(-1.0))
    stage_offs = []
    for t in range(3):
        h = 3 * s + t
        d = h // 2
        hv = h - 2 * d
        rest = d - 12 * q1
        img = rest // 6
        sig = rest - 6 * img
        src = (6 * c + sig) * N + hv * (HALF * 16)
        dst = halves[t]

        @pl.when(img == 0)
        def _copy_beta():
            pltpu.sync_copy(beta_hbm.at[pl.ds(src, HALF * 16)], dst)

        @pl.when(img == 1)
        def _copy_ground():
            pltpu.sync_copy(ground_hbm.at[pl.ds(src, HALF * 16)], dst)

        stage_offs.append(
            STAGE0 + c * 1536 + ((q1 * 6 + sig) * 4 + img * 2 + hv) * 32)

    def chunkv(ref, k):
        return sgn * ref[pl.ds(k * 16, 16)]

    def init_chain(ref):
        a = _sort16(chunkv(ref, 0), descending=False)
        b = _sort16(chunkv(ref, 1), descending=True)
        u = _sort16(jnp.maximum(a, b), descending=False)
        l = _sort16(jnp.minimum(a, b), descending=False)
        return u, l

    u0, l0 = init_chain(half_a)
    u1, l1 = init_chain(half_b)
    u2, l2 = init_chain(half_c)

    def body(k, carry):
        u0, l0, u1, l1, u2, l2 = carry
        u0, l0 = _merge_top(u0, l0, _sort16(chunkv(half_a, k),
                                            descending=True))
        u1, l1 = _merge_top(u1, l1, _sort16(chunkv(half_b, k),
                                            descending=True))
        u2, l2 = _merge_top(u2, l2, _sort16(chunkv(half_c, k),
                                            descending=True))
        return u0, l0, u1, l1, u2, l2

    u0, l0, u1, l1, u2, l2 = lax.fori_loop(
        2, HALF, body, (u0, l0, u1, l1, u2, l2))

    for off, (u, l) in zip(stage_offs, ((u0, l0), (u1, l1), (u2, l2))):
        s32_v[pl.ds(0, 16)] = u
        s32_v[pl.ds(16, 16)] = l
        pltpu.sync_copy(s32_v, out_hbm.at[pl.ds(off, 32)])

    plsc.subcore_barrier()

    @pl.when(s < 12)
    def _task():
        q = s // 6
        sgn2 = jnp.where(q == 0, np.float32(1.0), np.float32(-1.0))
        pltpu.sync_copy(out_hbm.at[pl.ds(STAGE0 + c * 1536 + s * 128, 128)],
                        loss12_v.at[pl.ds(0, 128)])
        ub, lb = _merge_sets(loss12_v[pl.ds(0, 16)], loss12_v[pl.ds(16, 16)],
                             loss12_v[pl.ds(32, 16)], loss12_v[pl.ds(48, 16)])
        ug, lg = _merge_sets(loss12_v[pl.ds(64, 16)], loss12_v[pl.ds(80, 16)],
                             loss12_v[pl.ds(96, 16)], loss12_v[pl.ds(112, 16)])

        idx_e = jnp.where(q == 0, 2 * iota, 2 * iota + 1)
        idx_s = jnp.where(q == 0, 2 * iota + 1, 2 * iota)
        s32_v[pl.ds(0, 16)] = _sort16(ub, descending=True)
        s32_v[pl.ds(16, 16)] = _sort16(lb, descending=True)
        de = sgn2 * plsc.load_gather(s32_v, [idx_e])
        dst = sgn2 * plsc.load_gather(s32_v, [idx_s])
        s32_v[pl.ds(0, 16)] = _sort16(ug, descending=True)
        s32_v[pl.ds(16, 16)] = _sort16(lg, descending=True)
        ge = sgn2 * plsc.load_gather(s32_v, [idx_e])
        gs = sgn2 * plsc.load_gather(s32_v, [idx_s])

        m = _valid_mask(de, dst)
        mg = _valid_mask(ge, gs)
        pen = (np.float32(1.0) - mg) * BIG

        used = jnp.zeros((16,), jnp.float32)
        acc = np.float32(0.0)
        one = np.float32(1.0)
        for i in range(K):
            e_i = de[i]
            s_i = dst[i]
            m_i = m[i]
            dx = e_i - ge
            dy = s_i - gs
            crow = dx * dx + dy * dy + pen + used * BIG
            mn = jnp.min(crow)
            j = plsc.all_reduce_ffs(crow == mn)
            oh = iota == j
            mg_j = jnp.sum(jnp.where(oh, mg, np.float32(0.0)))
            ge_j = jnp.sum(jnp.where(oh, ge, np.float32(0.0)))
            gs_j = jnp.sum(jnp.where(oh, gs, np.float32(0.0)))
            take = m_i * mg_j
            rm = (e_i + s_i) * np.float32(0.5)
            o_e = take * ge_j + (one - take) * rm
            o_s = take * gs_j + (one - take) * rm
            dd_e = (e_i - o_e) * m_i
            dd_s = (s_i - o_s) * m_i
            acc = acc + dd_e * dd_e + dd_s * dd_s
            used = used + jnp.where(oh, take, np.float32(0.0))

        xv = acc + np.float32(1e-12) + jnp.zeros((16,), jnp.float32)
        res_v[...] = _sqrt16(xv)
        pltpu.sync_copy(res_v, out_hbm.at[pl.ds(LOSS0 + (12 * c + s) * 16,
                                                16)])

    plsc.subcore_barrier()

    @pl.when(s == 0)
    def _reduce():
        pltpu.sync_copy(out_hbm.at[pl.ds(LOSS0 + c * 192, 192)], loss12_v)
        total = jnp.zeros((16,), jnp.float32)
        for w in range(12):
            total = total + loss12_v[pl.ds(w * 16, 16)]
        res_v[...] = total * np.float32(1.0 / 12.0)
        pltpu.sync_copy(res_v, out_hbm.at[pl.ds(c * 16, 16)])


@functools.partial(
    pl.kernel,
    out_type=jax.ShapeDtypeStruct((STAGE0 + 2 * 1536,), jnp.float32),
    mesh=plsc.VectorSubcoreMesh(core_axis_name="c", subcore_axis_name="s",
                                num_cores=2, num_subcores=16),
    compiler_params=pltpu.CompilerParams(needs_layout_passes=False),
    scratch_types=[
        pltpu.VMEM((HALF * 16,), jnp.float32),
        pltpu.VMEM((HALF * 16,), jnp.float32),
        pltpu.VMEM((HALF * 16,), jnp.float32),
        pltpu.VMEM((32,), jnp.float32),
        pltpu.VMEM((192,), jnp.float32),
        pltpu.VMEM((16,), jnp.float32),
    ],
)
def _toploss(beta_hbm, ground_hbm, out_hbm, half_a, half_b, half_c, s32_v,
             loss12_v, res_v):
    _toploss_body(beta_hbm, ground_hbm, out_hbm, half_a, half_b, half_c,
                  s32_v, loss12_v, res_v)


@jax.jit
def kernel(beta, ground):
    out = _toploss(beta.reshape(-1), ground.reshape(-1))
    return out[0] + out[16]
